# baseline (speedup 1.0000x reference)
import functools

import jax
import jax.numpy as jnp
from jax import lax
from jax.experimental import pallas as pl
from jax.experimental.pallas import tpu as pltpu
from jax.experimental.pallas import tpu_sc as plsc

V, D, B, L, OUT = 100000, 300, 1024, 200, 256
NC, NS = 2, 16
NW = NC * NS
R = 2 * B
ROWS_W = R // NW
LP = 208
G = 104
NCHUNK = LP // G
COUNTS = (G, L - G)
NCHUNKS_W = ROWS_W * NCHUNK
NVC = OUT // 16
MBLK = 800


def _pool_kernel(idx_hbm, tw_hbm, bias_hbm, out_hbm, idx_v, buf0, buf1,
                 bias_v, obuf_v, sem0, sem1):
  wid = lax.axis_index("s") * NC + lax.axis_index("c")
  idx_base = wid * (ROWS_W * LP)

  pltpu.sync_copy(idx_hbm.at[pl.ds(idx_base, ROWS_W * LP)], idx_v)
  pltpu.sync_copy(bias_hbm, bias_v)

  bufs = (buf0, buf1)
  sems = (sem0, sem1)

  def start_gather(flat_chunk, parity):
    pltpu.make_async_copy(
        tw_hbm.at[idx_v.at[pl.ds(flat_chunk * G, G)]],
        bufs[parity], sems[parity]).start()

  start_gather(jnp.int32(0), 0)

  inv_l = jnp.float32(1.0 / L)

  def row_body(b, carry):
    acc = [jnp.zeros((16,), jnp.float32)] * NVC
    for k in range(NCHUNK):
      flat = b * NCHUNK + k
      p = k % 2
      pltpu.make_async_copy(
          tw_hbm.at[idx_v.at[pl.ds(flat * G, G)]],
          bufs[p], sems[p]).wait()

      @pl.when(flat + 1 < NCHUNKS_W)
      def _():
        start_gather(flat + 1, (k + 1) % 2)

      buf = bufs[p]

      def accum_body(l, acc):
        return tuple(acc[c] + buf[l, pl.ds(c * 16, 16)] for c in range(NVC))

      acc = lax.fori_loop(0, COUNTS[k], accum_body, tuple(acc))

    for c in range(NVC):
      obuf_v[b, pl.ds(c * 16, 16)] = (
          acc[c] * inv_l + bias_v[pl.ds(c * 16, 16)])
    return carry

  lax.fori_loop(0, ROWS_W, row_body, jnp.int32(0))
  pltpu.sync_copy(obuf_v, out_hbm.at[pl.ds(wid * ROWS_W, ROWS_W)])


def _mm_kernel(x_ref, wt_ref, o_ref):
  o_ref[...] = jnp.dot(x_ref[...], wt_ref[...],
                       preferred_element_type=jnp.float32)


@jax.jit
def _fused(idx_flat, table, wt, bias):
  tablew = pl.pallas_call(
      _mm_kernel,
      grid=(V // MBLK,),
      in_specs=[
          pl.BlockSpec((MBLK, D), lambda i: (i, 0)),
          pl.BlockSpec((D, OUT), lambda i: (0, 0)),
      ],
      out_specs=pl.BlockSpec((MBLK, OUT), lambda i: (i, 0)),
      out_shape=jax.ShapeDtypeStruct((V, OUT), jnp.float32),
  )(table, wt)

  mesh = plsc.VectorSubcoreMesh(core_axis_name="c", subcore_axis_name="s",
                                num_cores=NC, num_subcores=NS)
  return pl.kernel(
      _pool_kernel,
      out_type=jax.ShapeDtypeStruct((R, OUT), jnp.float32),
      mesh=mesh,
      compiler_params=pltpu.CompilerParams(use_tc_tiling_on_sc=False),
      scratch_types=[
          pltpu.VMEM((ROWS_W * LP,), jnp.int32),
          pltpu.VMEM((G, OUT), jnp.float32),
          pltpu.VMEM((G, OUT), jnp.float32),
          pltpu.VMEM((OUT,), jnp.float32),
          pltpu.VMEM((ROWS_W, OUT), jnp.float32),
          pltpu.SemaphoreType.DMA,
          pltpu.SemaphoreType.DMA,
      ],
  )(idx_flat, tablew, bias)


def kernel(inputs_1, inputs_2, table, W, b):
  idx = jnp.concatenate([inputs_1, inputs_2], axis=0)
  idx_flat = jnp.pad(idx, ((0, 0), (0, LP - L))).reshape(-1)
  out = _fused(idx_flat, table, W.T, b)
  return out[:B], out[B:]


# speedup vs baseline: 1.0712x; 1.0712x over previous
import functools

import numpy as np
import jax
import jax.numpy as jnp
from jax import lax
from jax.experimental import pallas as pl
from jax.experimental.pallas import tpu as pltpu
from jax.experimental.pallas import tpu_sc as plsc

V, D, B, L, OUT = 100000, 300, 1024, 200, 256
NC, NS = 2, 16
NW = NC * NS
R = 2 * B
ROWS_W = R // NW
LP = 208
G = 104
NCHUNK = LP // G
COUNTS = (G, L - G)
NCHUNKS_W = ROWS_W * NCHUNK
NJ = OUT // 32
MBLK = 800

_PERM = np.empty((OUT,), np.int32)
for _j in range(NJ):
  _PERM[32 * _j + 0:32 * _j + 32:2] = np.arange(16) + 32 * _j
  _PERM[32 * _j + 1:32 * _j + 32:2] = np.arange(16) + 32 * _j + 16


def _pool_kernel(idx_hbm, tw_hbm, bias_hbm, out_hbm, idx_v, buf0, buf1,
                 bias_v, obuf_v, sem0, sem1):
  wid = lax.axis_index("s") * NC + lax.axis_index("c")
  idx_base = wid * (ROWS_W * LP)

  pltpu.sync_copy(idx_hbm.at[pl.ds(idx_base, ROWS_W * LP)], idx_v)
  pltpu.sync_copy(bias_hbm, bias_v)

  bufs = (buf0, buf1)
  sems = (sem0, sem1)

  def start_gather(flat_chunk, parity):
    pltpu.make_async_copy(
        tw_hbm.at[idx_v.at[pl.ds(flat_chunk * G, G)]],
        bufs[parity], sems[parity]).start()

  start_gather(jnp.int32(0), 0)

  inv_l = jnp.float32(1.0 / L)

  def row_body(b, carry):
    acc = [jnp.zeros((16,), jnp.float32)] * (2 * NJ)
    for k in range(NCHUNK):
      flat = b * NCHUNK + k
      p = k % 2
      pltpu.make_async_copy(
          tw_hbm.at[idx_v.at[pl.ds(flat * G, G)]],
          bufs[p], sems[p]).wait()

      @pl.when(flat + 1 < NCHUNKS_W)
      def _():
        start_gather(flat + 1, (k + 1) % 2)

      buf = bufs[p]

      def accum_body(l, acc):
        acc = list(acc)
        for j in range(NJ):
          v = plsc.bitcast(buf[l, pl.ds(32 * j, 32)], jnp.int32)
          lo = plsc.bitcast(lax.shift_left(v, 16), jnp.float32)
          hi = plsc.bitcast(
              lax.bitwise_and(v, jnp.int32(-65536)), jnp.float32)
          acc[2 * j] = acc[2 * j] + lo
          acc[2 * j + 1] = acc[2 * j + 1] + hi
        return tuple(acc)

      acc = lax.fori_loop(0, COUNTS[k], accum_body, tuple(acc))

    for c in range(2 * NJ):
      obuf_v[b, pl.ds(c * 16, 16)] = (
          acc[c] * inv_l + bias_v[pl.ds(c * 16, 16)])
    return carry

  lax.fori_loop(0, ROWS_W, row_body, jnp.int32(0))
  pltpu.sync_copy(obuf_v, out_hbm.at[pl.ds(wid * ROWS_W, ROWS_W)])


def _mm_kernel(x_ref, wt_ref, o_ref):
  o_ref[...] = jnp.dot(x_ref[...], wt_ref[...],
                       preferred_element_type=jnp.float32).astype(jnp.bfloat16)


@jax.jit
def _fused(idx_flat, table, wt, bias):
  tablew = pl.pallas_call(
      _mm_kernel,
      grid=(V // MBLK,),
      in_specs=[
          pl.BlockSpec((MBLK, D), lambda i: (i, 0)),
          pl.BlockSpec((D, OUT), lambda i: (0, 0)),
      ],
      out_specs=pl.BlockSpec((MBLK, OUT), lambda i: (i, 0)),
      out_shape=jax.ShapeDtypeStruct((V, OUT), jnp.bfloat16),
  )(table, wt)

  mesh = plsc.VectorSubcoreMesh(core_axis_name="c", subcore_axis_name="s",
                                num_cores=NC, num_subcores=NS)
  return pl.kernel(
      _pool_kernel,
      out_type=jax.ShapeDtypeStruct((R, OUT), jnp.float32),
      mesh=mesh,
      compiler_params=pltpu.CompilerParams(use_tc_tiling_on_sc=False,
                                           needs_layout_passes=False),
      scratch_types=[
          pltpu.VMEM((ROWS_W * LP,), jnp.int32),
          pltpu.VMEM((G, OUT), jnp.bfloat16),
          pltpu.VMEM((G, OUT), jnp.bfloat16),
          pltpu.VMEM((OUT,), jnp.float32),
          pltpu.VMEM((ROWS_W, OUT), jnp.float32),
          pltpu.SemaphoreType.DMA,
          pltpu.SemaphoreType.DMA,
      ],
  )(idx_flat, tablew, bias)


def kernel(inputs_1, inputs_2, table, W, b):
  idx = jnp.concatenate([inputs_1, inputs_2], axis=0)
  idx_flat = jnp.pad(idx, ((0, 0), (0, LP - L))).reshape(-1)
  perm = jnp.asarray(_PERM)
  out = _fused(idx_flat, table, W.T[:, perm], b)
  return out[:B], out[B:]
